# baseline (speedup 1.0000x reference)
import functools

import jax
import jax.numpy as jnp
from jax import lax
from jax.experimental import pallas as pl
from jax.experimental.pallas import tpu as pltpu
from jax.experimental.pallas import tpu_sc as plsc

N_NODES = 10000
D = 128
N_EDGES = 320000
EPS = 1e-5

NC = 2
NS = 16
DH = D // NC
EPW = N_EDGES // NS
K = 80
NCHUNK = EPW // K
NBUF = 5
N_PAD = 10240
ROWS_PER_SUB = N_PAD // NS
DEG_W = 16
LPR = K // 16

_mesh = plsc.VectorSubcoreMesh(core_axis_name="c", subcore_axis_name="s")


@functools.partial(
    pl.kernel,
    mesh=_mesh,
    compiler_params=pltpu.CompilerParams(use_tc_tiling_on_sc=False),
    out_type=[
        jax.ShapeDtypeStruct((N_PAD, D), jnp.float32),
        jax.ShapeDtypeStruct((N_PAD, NC * DEG_W), jnp.float32),
    ],
    scratch_types=[
        pltpu.VMEM((NCHUNK, K), jnp.int32),
        pltpu.VMEM((NCHUNK, K), jnp.int32),
        pltpu.VMEM((K, DEG_W), jnp.float32),
        *[pltpu.VMEM((K, DH), jnp.float32) for _ in range(NBUF)],
        pltpu.VMEM_SHARED((N_PAD, DH), jnp.float32),
        pltpu.VMEM_SHARED((N_PAD, DEG_W), jnp.float32),
        *[pltpu.SemaphoreType.DMA for _ in range(NBUF)],
    ],
)
def _sc_aggregate(x2_hbm, esrc_hbm, edst_hbm, zacc_hbm, zdeg_hbm,
                  acc_out_hbm, deg_out_hbm,
                  src_v, dst_v, ones_v, *rest):
    bufs = rest[:NBUF]
    acc_s = rest[NBUF]
    deg_s = rest[NBUF + 1]
    sems = rest[NBUF + 2:NBUF + 2 + NBUF]

    c = lax.axis_index("c")
    s = lax.axis_index("s")

    pltpu.sync_copy(esrc_hbm.at[c, s], src_v)
    pltpu.sync_copy(edst_hbm.at[s], dst_v)

    one16 = jnp.ones((16,), jnp.float32)
    for i in range(K):
        ones_v[i, :] = one16

    row0 = s * ROWS_PER_SUB
    pltpu.sync_copy(zacc_hbm.at[pl.ds(row0, ROWS_PER_SUB)],
                    acc_s.at[pl.ds(row0, ROWS_PER_SUB)])
    pltpu.sync_copy(zdeg_hbm.at[pl.ds(row0, ROWS_PER_SUB)],
                    deg_s.at[pl.ds(row0, ROWS_PER_SUB)])
    plsc.subcore_barrier()

    def gather_start(j, b):
        pltpu.make_async_copy(x2_hbm.at[src_v.at[j]], bufs[b], sems[b]).start()

    def gather_wait(j, b):
        pltpu.make_async_copy(x2_hbm.at[src_v.at[j]], bufs[b], sems[b]).wait()

    def scatter(j, b):
        pltpu.sync_copy(bufs[b], acc_s.at[dst_v.at[j]], add=True)
        do_deg = jnp.logical_xor(j < NCHUNK // 2, c == 1)

        @pl.when(do_deg)
        def _():
            pltpu.sync_copy(ones_v, deg_s.at[dst_v.at[j]], add=True)

    for b in range(NBUF):
        gather_start(b, b)

    def outer(o, carry):
        base = o * NBUF
        for b in range(NBUF):
            j = base + b
            gather_wait(j, b)
            scatter(j, b)
            gather_start(j + NBUF, b)
        return carry

    lax.fori_loop(0, NCHUNK // NBUF - 1, outer, 0)

    base = NCHUNK - NBUF
    for b in range(NBUF):
        gather_wait(base + b, b)
        scatter(base + b, b)

    plsc.subcore_barrier()

    pltpu.sync_copy(acc_s.at[pl.ds(row0, ROWS_PER_SUB)],
                    acc_out_hbm.at[pl.ds(row0, ROWS_PER_SUB),
                                   pl.ds(c * DH, DH)])
    pltpu.sync_copy(deg_s.at[pl.ds(row0, ROWS_PER_SUB)],
                    deg_out_hbm.at[pl.ds(row0, ROWS_PER_SUB),
                                   pl.ds(c * DEG_W, DEG_W)])


def _dense_body(p_ref, dp_ref, w_ref, b_ref, g_ref, be_ref, al_ref, o_ref):
    acc = p_ref[:N_NODES]
    deg = dp_ref[:N_NODES, 0:1] + dp_ref[:N_NODES, DEG_W:DEG_W + 1]
    agg = acc / jnp.maximum(deg, 1.0)
    h = jnp.dot(agg, w_ref[...], preferred_element_type=jnp.float32)
    h = h + b_ref[...]
    mean = jnp.mean(h, axis=0, keepdims=True)
    h_c = h - al_ref[...] * mean
    var = jnp.mean(h_c * h_c, axis=0, keepdims=True)
    out = g_ref[...] * (h_c * lax.rsqrt(var + EPS)) + be_ref[...]
    o_ref[...] = jnp.maximum(out, 0.0)


_dense = pl.pallas_call(
    _dense_body,
    out_shape=jax.ShapeDtypeStruct((N_NODES, D), jnp.float32),
)


def kernel(x, edge_index, W, b, gamma, beta, alpha):
    x2 = x.reshape(NC * N_NODES, DH)
    ei32 = edge_index.astype(jnp.int32)
    esrc = (2 * ei32[0][None, :]
            + jnp.arange(NC, dtype=jnp.int32)[:, None]).reshape(
        NC, NS, NCHUNK, K)
    edst = ei32[1].reshape(NS, NCHUNK, K)
    zacc = jnp.zeros((N_PAD, DH), jnp.float32)
    zdeg = jnp.zeros((N_PAD, DEG_W), jnp.float32)
    acc_p, deg_p = _sc_aggregate(x2, esrc, edst, zacc, zdeg)
    return _dense(acc_p, deg_p, W,
                  b.reshape(1, D), gamma.reshape(1, D),
                  beta.reshape(1, D), alpha.reshape(1, D))


# speedup vs baseline: 1.1582x; 1.1582x over previous
import functools

import jax
import jax.numpy as jnp
from jax import lax
from jax.experimental import pallas as pl
from jax.experimental.pallas import tpu as pltpu
from jax.experimental.pallas import tpu_sc as plsc

N_NODES = 10000
D = 128
N_EDGES = 320000
EPS = 1e-5

NC = 2
NS = 16
DH = D // NC
EPW = N_EDGES // NS
K = 80
NCHUNK = EPW // K
NBUF = 5
N_PAD = 10240
ROWS_PER_SUB = N_PAD // NS
DEG_W = 16
LPR = K // 16

_mesh = plsc.VectorSubcoreMesh(core_axis_name="c", subcore_axis_name="s")


@functools.partial(
    pl.kernel,
    mesh=_mesh,
    compiler_params=pltpu.CompilerParams(use_tc_tiling_on_sc=False),
    out_type=[
        jax.ShapeDtypeStruct((N_PAD, D), jnp.float32),
        jax.ShapeDtypeStruct((N_PAD, NC * DEG_W), jnp.float32),
    ],
    scratch_types=[
        pltpu.VMEM((NCHUNK, K), jnp.int32),
        pltpu.VMEM((NCHUNK, K), jnp.int32),
        pltpu.VMEM((K, DEG_W), jnp.float32),
        *[pltpu.VMEM((K, DH), jnp.float32) for _ in range(NBUF)],
        pltpu.VMEM_SHARED((N_PAD, DH), jnp.float32),
        pltpu.VMEM_SHARED((N_PAD, DEG_W), jnp.float32),
        *[pltpu.SemaphoreType.DMA for _ in range(NBUF)],
    ],
)
def _sc_aggregate(x2_hbm, ei_hbm, zacc_hbm, zdeg_hbm,
                  acc_out_hbm, deg_out_hbm,
                  src_v, dst_v, ones_v, *rest):
    bufs = rest[:NBUF]
    acc_s = rest[NBUF]
    deg_s = rest[NBUF + 1]
    sems = rest[NBUF + 2:NBUF + 2 + NBUF]

    c = lax.axis_index("c")
    s = lax.axis_index("s")

    row0 = s * ROWS_PER_SUB
    src_cp = pltpu.make_async_copy(ei_hbm.at[0, s], src_v, sems[0])
    dst_cp = pltpu.make_async_copy(ei_hbm.at[1, s], dst_v, sems[1])
    zacc_cp = pltpu.make_async_copy(zacc_hbm.at[pl.ds(row0, ROWS_PER_SUB)],
                                    acc_s.at[pl.ds(row0, ROWS_PER_SUB)],
                                    sems[2])
    zdeg_cp = pltpu.make_async_copy(zdeg_hbm.at[pl.ds(row0, ROWS_PER_SUB)],
                                    deg_s.at[pl.ds(row0, ROWS_PER_SUB)],
                                    sems[3])
    for cp in (src_cp, dst_cp, zacc_cp, zdeg_cp):
        cp.start()

    one16 = jnp.ones((16,), jnp.float32)
    for i in range(K):
        ones_v[i, :] = one16

    two = jnp.full((16,), 2, jnp.int32)
    coff = jnp.full((16,), 1, jnp.int32) * c
    src_cp.wait()

    def adj(r, carry):
        for q in range(LPR):
            v = src_v[r, pl.ds(q * 16, 16)]
            src_v[r, pl.ds(q * 16, 16)] = v * two + coff
        return carry

    lax.fori_loop(0, NCHUNK, adj, 0)

    dst_cp.wait()
    zacc_cp.wait()
    zdeg_cp.wait()
    plsc.subcore_barrier()

    def gather_start(j, b):
        pltpu.make_async_copy(x2_hbm.at[src_v.at[j]], bufs[b], sems[b]).start()

    def gather_wait(j, b):
        pltpu.make_async_copy(x2_hbm.at[src_v.at[j]], bufs[b], sems[b]).wait()

    def scatter(j, b):
        pltpu.sync_copy(bufs[b], acc_s.at[dst_v.at[j]], add=True)
        do_deg = jnp.logical_xor(j < NCHUNK // 2, c == 1)

        @pl.when(do_deg)
        def _():
            pltpu.sync_copy(ones_v, deg_s.at[dst_v.at[j]], add=True)

    for b in range(NBUF):
        gather_start(b, b)

    def outer(o, carry):
        base = o * NBUF
        for b in range(NBUF):
            j = base + b
            gather_wait(j, b)
            scatter(j, b)
            gather_start(j + NBUF, b)
        return carry

    lax.fori_loop(0, NCHUNK // NBUF - 1, outer, 0)

    base = NCHUNK - NBUF
    for b in range(NBUF):
        gather_wait(base + b, b)
        scatter(base + b, b)

    plsc.subcore_barrier()

    out_cp = pltpu.make_async_copy(
        acc_s.at[pl.ds(row0, ROWS_PER_SUB)],
        acc_out_hbm.at[pl.ds(row0, ROWS_PER_SUB), pl.ds(c * DH, DH)],
        sems[0])
    deg_cp = pltpu.make_async_copy(
        deg_s.at[pl.ds(row0, ROWS_PER_SUB)],
        deg_out_hbm.at[pl.ds(row0, ROWS_PER_SUB), pl.ds(c * DEG_W, DEG_W)],
        sems[1])
    out_cp.start()
    deg_cp.start()
    out_cp.wait()
    deg_cp.wait()


def _dense_body(p_ref, dp_ref, w_ref, b_ref, g_ref, be_ref, al_ref, o_ref):
    acc = p_ref[:N_NODES]
    deg = dp_ref[:N_NODES, 0:1] + dp_ref[:N_NODES, DEG_W:DEG_W + 1]
    agg = acc / jnp.maximum(deg, 1.0)
    h = jnp.dot(agg, w_ref[...], preferred_element_type=jnp.float32)
    h = h + b_ref[...]
    mean = jnp.mean(h, axis=0, keepdims=True)
    h_c = h - al_ref[...] * mean
    var = jnp.mean(h_c * h_c, axis=0, keepdims=True)
    out = g_ref[...] * (h_c * lax.rsqrt(var + EPS)) + be_ref[...]
    o_ref[...] = jnp.maximum(out, 0.0)


_dense = pl.pallas_call(
    _dense_body,
    out_shape=jax.ShapeDtypeStruct((N_NODES, D), jnp.float32),
)


def kernel(x, edge_index, W, b, gamma, beta, alpha):
    x2 = x.reshape(NC * N_NODES, DH)
    ei = edge_index.astype(jnp.int32).reshape(2, NS, NCHUNK, K)
    zacc = jnp.zeros((N_PAD, DH), jnp.float32)
    zdeg = jnp.zeros((N_PAD, DEG_W), jnp.float32)
    acc_p, deg_p = _sc_aggregate(x2, ei, zacc, zdeg)
    return _dense(acc_p, deg_p, W,
                  b.reshape(1, D), gamma.reshape(1, D),
                  beta.reshape(1, D), alpha.reshape(1, D))
